# baseline (speedup 1.0000x reference)
import functools

import jax
import jax.numpy as jnp
from jax import lax
from jax.experimental import pallas as pl
from jax.experimental.pallas import tpu as pltpu
from jax.experimental.pallas import tpu_sc as plsc

D_MODEL = 1024
MAX_LEN = 8192
SEQ_LEN = 32768

NUM_CORES = 2
NUM_SUBCORES = 16
NUM_WORKERS = NUM_CORES * NUM_SUBCORES
B_PER_W = SEQ_LEN // NUM_WORKERS
CHUNK = 16
NCHUNK = B_PER_W // CHUNK


def _sc_gather(table, positions):
    mesh = plsc.VectorSubcoreMesh(
        core_axis_name="c", subcore_axis_name="s",
        num_cores=NUM_CORES, num_subcores=NUM_SUBCORES)

    @functools.partial(
        pl.kernel,
        mesh=mesh,
        out_type=jax.ShapeDtypeStruct((SEQ_LEN, D_MODEL), jnp.float32),
        scratch_types=[
            pltpu.VMEM((B_PER_W,), jnp.int32),
            [pltpu.VMEM((CHUNK, D_MODEL), jnp.float32) for _ in range(2)],
            [pltpu.VMEM_SHARED((NUM_SUBCORES, CHUNK, D_MODEL), jnp.float32)
             for _ in range(2)],
            [pltpu.SemaphoreType.DMA for _ in range(2)],
            [pltpu.SemaphoreType.DMA for _ in range(2)],
            [pltpu.SemaphoreType.DMA for _ in range(2)],
        ],
    )
    def k(tab_hbm, idx_hbm, out_hbm, idx_v, bufs, stages, gsems, csems, wsems):
        sid = lax.axis_index("s")
        wid = sid * NUM_CORES + lax.axis_index("c")
        base = wid * B_PER_W
        pltpu.sync_copy(idx_hbm.at[pl.ds(base, B_PER_W)], idx_v)

        def start_gather(j, b):
            pltpu.async_copy(
                tab_hbm.at[idx_v.at[pl.ds(j * CHUNK, CHUNK)]],
                bufs[b], gsems[b])

        def wait_gather(b):
            pltpu.make_async_copy(
                tab_hbm.at[pl.ds(0, CHUNK)], bufs[b], gsems[b]).wait()

        def start_copy(b):
            pltpu.async_copy(bufs[b], stages[b].at[sid], csems[b])

        def wait_copy(b):
            pltpu.make_async_copy(
                bufs[b], stages[b].at[sid], csems[b]).wait()

        def start_write(j, b):
            pltpu.async_copy(
                stages[b].at[sid], out_hbm.at[pl.ds(base + j * CHUNK, CHUNK)],
                wsems[b])

        def wait_write(b):
            pltpu.make_async_copy(
                stages[b].at[sid], out_hbm.at[pl.ds(base, CHUNK)],
                wsems[b]).wait()

        start_gather(0, 0)

        @pl.loop(0, NCHUNK, step=2)
        def _(i0):
            for bb in range(2):
                i = i0 + bb
                ob = 1 - bb
                wait_gather(bb)
                @pl.when(i >= 2)
                def _():
                    wait_write(bb)
                start_copy(bb)
                @pl.when(i >= 1)
                def _():
                    wait_copy(ob)
                    start_write(i - 1, ob)
                @pl.when(i + 1 < NCHUNK)
                def _():
                    start_gather(i + 1, ob)

        wait_copy(1)
        start_write(NCHUNK - 1, 1)
        wait_write(0)
        wait_write(1)

    return k(table, positions)


def kernel(positions, encoding_table):
    return _sc_gather(encoding_table, positions.astype(jnp.int32))


# speedup vs baseline: 1.1142x; 1.1142x over previous
import functools

import jax
import jax.numpy as jnp
from jax import lax
from jax.experimental import pallas as pl
from jax.experimental.pallas import tpu as pltpu
from jax.experimental.pallas import tpu_sc as plsc

D_MODEL = 1024
MAX_LEN = 8192
SEQ_LEN = 32768

NUM_CORES = 2
NUM_SUBCORES = 16
NUM_WORKERS = NUM_CORES * NUM_SUBCORES
B_PER_W = SEQ_LEN // NUM_WORKERS
CHUNK = 32
NCHUNK = B_PER_W // CHUNK
NBUF = 3
NMAIN = (NCHUNK // NBUF) * NBUF


def _sc_gather(table, positions):
    mesh = plsc.VectorSubcoreMesh(
        core_axis_name="c", subcore_axis_name="s",
        num_cores=NUM_CORES, num_subcores=NUM_SUBCORES)

    @functools.partial(
        pl.kernel,
        mesh=mesh,
        out_type=jax.ShapeDtypeStruct((SEQ_LEN, D_MODEL), jnp.float32),
        scratch_types=[
            pltpu.VMEM((B_PER_W,), jnp.int32),
            [pltpu.VMEM((CHUNK, D_MODEL), jnp.float32) for _ in range(NBUF)],
            [pltpu.SemaphoreType.DMA for _ in range(NBUF)],
            [pltpu.SemaphoreType.DMA for _ in range(NBUF)],
        ],
    )
    def k(tab_hbm, idx_hbm, out_hbm, idx_v, bufs, gsems, wsems):
        wid = lax.axis_index("s") * NUM_CORES + lax.axis_index("c")
        base = wid * B_PER_W
        pltpu.sync_copy(idx_hbm.at[pl.ds(base, B_PER_W)], idx_v)

        def start_gather(j, b):
            pltpu.async_copy(
                tab_hbm.at[idx_v.at[pl.ds(j * CHUNK, CHUNK)]],
                bufs[b], gsems[b])

        def wait_gather(b):
            pltpu.make_async_copy(
                tab_hbm.at[pl.ds(0, CHUNK)], bufs[b], gsems[b]).wait()

        def start_write(j, b):
            pltpu.async_copy(
                bufs[b], out_hbm.at[pl.ds(base + j * CHUNK, CHUNK)],
                wsems[b])

        def wait_write(b):
            pltpu.make_async_copy(
                bufs[b], out_hbm.at[pl.ds(base, CHUNK)], wsems[b]).wait()

        start_gather(0, 0)

        @pl.loop(0, NMAIN, step=NBUF)
        def _(i0):
            for bb in range(NBUF):
                i = i0 + bb
                nb = (bb + 1) % NBUF
                @pl.when(i + 1 >= NBUF)
                def _():
                    wait_write(nb)
                start_gather(i + 1, nb)
                wait_gather(bb)
                start_write(i, bb)

        for i in range(NMAIN, NCHUNK):
            bb = i % NBUF
            nb = (i + 1) % NBUF
            if i + 1 < NCHUNK:
                wait_write(nb)
                start_gather(i + 1, nb)
            wait_gather(bb)
            start_write(i, bb)

        for bb in range(NBUF):
            wait_write(bb)

    return k(table, positions)


def kernel(positions, encoding_table):
    return _sc_gather(encoding_table, positions.astype(jnp.int32))


# speedup vs baseline: 1.1196x; 1.0049x over previous
import functools

import jax
import jax.numpy as jnp
from jax import lax
from jax.experimental import pallas as pl
from jax.experimental.pallas import tpu as pltpu
from jax.experimental.pallas import tpu_sc as plsc

D_MODEL = 1024
MAX_LEN = 8192
SEQ_LEN = 32768

NUM_CORES = 2
NUM_SUBCORES = 16
NUM_WORKERS = NUM_CORES * NUM_SUBCORES
B_PER_W = SEQ_LEN // NUM_WORKERS
CHUNK = 16
NCHUNK = B_PER_W // CHUNK
NBUF = 4


def _sc_gather(table, positions):
    mesh = plsc.VectorSubcoreMesh(
        core_axis_name="c", subcore_axis_name="s",
        num_cores=NUM_CORES, num_subcores=NUM_SUBCORES)

    @functools.partial(
        pl.kernel,
        mesh=mesh,
        out_type=jax.ShapeDtypeStruct((SEQ_LEN, D_MODEL), jnp.float32),
        scratch_types=[
            pltpu.VMEM((B_PER_W,), jnp.int32),
            [pltpu.VMEM((CHUNK, D_MODEL), jnp.float32) for _ in range(NBUF)],
            [pltpu.SemaphoreType.DMA for _ in range(NBUF)],
            [pltpu.SemaphoreType.DMA for _ in range(NBUF)],
        ],
    )
    def k(tab_hbm, idx_hbm, out_hbm, idx_v, bufs, gsems, wsems):
        wid = lax.axis_index("s") * NUM_CORES + lax.axis_index("c")
        base = wid * B_PER_W
        pltpu.sync_copy(idx_hbm.at[pl.ds(base, B_PER_W)], idx_v)

        def start_gather(j, b):
            pltpu.async_copy(
                tab_hbm.at[idx_v.at[pl.ds(j * CHUNK, CHUNK)]],
                bufs[b], gsems[b])

        def drain_gather(b):
            pltpu.make_async_copy(
                tab_hbm.at[pl.ds(0, CHUNK)], bufs[b], gsems[b]).wait()

        def drain_write(b):
            pltpu.make_async_copy(
                bufs[b], out_hbm.at[pl.ds(base, CHUNK)], wsems[b]).wait()

        start_gather(0, 0)

        @pl.loop(0, NCHUNK, step=NBUF)
        def _(i0):
            for bb in range(NBUF):
                i = i0 + bb
                nb = (bb + 1) % NBUF
                @pl.when(i + 1 < NCHUNK)
                def _():
                    @pl.when(i + 1 >= NBUF)
                    def _():
                        drain_write(nb)
                    start_gather(i + 1, nb)
                drain_gather(bb)
                pltpu.async_copy(
                    bufs[bb], out_hbm.at[pl.ds(base + i * CHUNK, CHUNK)],
                    wsems[bb])

        for bb in range(NBUF):
            drain_write(bb)

    return k(table, positions)


def kernel(positions, encoding_table):
    return _sc_gather(encoding_table, positions.astype(jnp.int32))
